# baseline (speedup 1.0000x reference)
import functools
import numpy as np
import jax
import jax.numpy as jnp
from jax.experimental import pallas as pl
from jax.experimental.pallas import tpu as pltpu

_NB = 256


def _fused_body(k_sel, CC, base_ref, hp_ref, s_ref, W1_ref, b1_ref, W2_ref,
                b2_ref, Wg1_ref, bg1_ref, wg2t_ref, bg2_ref, out_ref):
    ib = pl.program_id(1)
    D = base_ref.shape[2]

    base = base_ref[0]
    H = hp_ref[0].astype(jnp.bfloat16)
    ni = _NB // 32
    H5 = H.reshape(ni, 2, 32, 2, D)
    pre = b1_ref[0]
    for p in range(2):
        for q in range(2):
            hpq = H5[:, p, :, q, :].reshape(_NB, D)
            Wpq = W1_ref[(2 * p + q) * D:(2 * p + q + 1) * D]
            pre = pre + jnp.dot(hpq, Wpq, preferred_element_type=jnp.float32)
    h = pre * 0.5 * (1.0 + jax.lax.erf(pre * np.float32(1.0 / np.sqrt(2.0))))
    refined = (jnp.dot(h.astype(jnp.bfloat16), W2_ref[...],
                       preferred_element_type=jnp.float32) + b2_ref[0])

    sblk = s_ref[0, 0, pl.ds(ib * _NB, _NB)]
    sblk_c = sblk[:, None]

    gi = (jnp.dot(base.astype(jnp.bfloat16), Wg1_ref[0:D],
                  preferred_element_type=jnp.float32)
          + jnp.dot(refined.astype(jnp.bfloat16), Wg1_ref[D:2 * D],
                    preferred_element_type=jnp.float32)
          + sblk_c * Wg1_ref[2 * D:2 * D + 1]
          + bg1_ref[0])
    g = gi * jax.nn.sigmoid(gi)
    gate = jax.nn.sigmoid(
        jnp.sum(g * wg2t_ref[0][None, :], axis=1, keepdims=True) + bg2_ref[0, 0])

    gidx = ib * _NB + jax.lax.broadcasted_iota(jnp.int32, (_NB, 1), 0)
    nchunk = CC // 128

    def body(c, acc):
        sc = s_ref[0, 0, pl.ds(c * 128, 128)][None, :]
        cidx = c * 128 + jax.lax.broadcasted_iota(jnp.int32, (1, 128), 1)
        beats = (sc > sblk_c) | ((sc == sblk_c) & (cidx < gidx))
        return acc + jnp.sum(beats.astype(jnp.float32), axis=1, keepdims=True)

    rank = jax.lax.fori_loop(0, nchunk, body, jnp.zeros((_NB, 1), jnp.float32))
    mask = (rank < np.float32(k_sel)).astype(jnp.float32)

    out_ref[0] = base + mask * gate * (refined - base)


def kernel(base_tokens, highres_tokens, selection_scores, W1, b1, W2, b2,
           Wg1, bg1, Wg2, bg2):
    B, CC, D = base_tokens.shape
    k_sel = max(1, int(round(CC * 0.15)))

    s3 = selection_scores.reshape(B, 1, CC)
    W1h = W1.astype(jnp.bfloat16)
    W2h = W2.astype(jnp.bfloat16)
    Wg1h = Wg1.astype(jnp.bfloat16)
    b1r = b1.reshape(1, -1)
    b2r = b2.reshape(1, -1)
    bg1r = bg1.reshape(1, -1)
    wg2t = Wg2.reshape(1, -1)
    bg2r = bg2.reshape(1, 1)

    nblk = CC // _NB
    grid = (B, nblk)

    full = lambda shape: pl.BlockSpec(shape, lambda b, i: (0,) * len(shape))

    out = pl.pallas_call(
        functools.partial(_fused_body, k_sel, CC),
        grid=grid,
        in_specs=[
            pl.BlockSpec((1, _NB, D), lambda b, i: (b, i, 0)),
            pl.BlockSpec((1, 4 * _NB, D), lambda b, i: (b, i, 0)),
            pl.BlockSpec((1, 1, CC), lambda b, i: (b, 0, 0)),
            full((4 * D, W1.shape[1])),
            full((1, b1.shape[0])),
            full(W2.shape),
            full((1, b2.shape[0])),
            full(Wg1.shape),
            full((1, bg1.shape[0])),
            full((1, Wg2.shape[0])),
            full((1, 1)),
        ],
        out_specs=pl.BlockSpec((1, _NB, D), lambda b, i: (b, i, 0)),
        out_shape=jax.ShapeDtypeStruct((B, CC, D), jnp.float32),
        compiler_params=pltpu.CompilerParams(
            dimension_semantics=("parallel", "parallel")),
    )(base_tokens, highres_tokens, s3, W1h, b1r, W2h, b2r, Wg1h, bg1r, wg2t,
      bg2r)
    return out


# speedup vs baseline: 1.3712x; 1.3712x over previous
import functools
import numpy as np
import jax
import jax.numpy as jnp
from jax.experimental import pallas as pl
from jax.experimental.pallas import tpu as pltpu

_NB = 256


def _fused_body(k_sel, CC, base_ref, hr_ref, s_ref, W1_ref, b1_ref, W2_ref,
                b2_ref, Wg1_ref, bg1_ref, wg2t_ref, bg2_ref, out_ref):
    ib = pl.program_id(1)
    D = base_ref.shape[1]

    bT = base_ref[0]
    base = bT.T
    H = hr_ref[0].T
    ni = _NB // 32
    H5 = H.reshape(ni, 2, 32, 2, D)
    pre = b1_ref[0]
    for p in range(2):
        for q in range(2):
            hpq = H5[:, p, :, q, :].reshape(_NB, D)
            Wpq = W1_ref[(2 * p + q) * D:(2 * p + q + 1) * D]
            pre = pre + jnp.dot(hpq, Wpq, preferred_element_type=jnp.float32)
    h = pre * 0.5 * (1.0 + jax.lax.erf(pre * np.float32(1.0 / np.sqrt(2.0))))
    refined = jnp.dot(h, W2_ref[...],
                      preferred_element_type=jnp.float32) + b2_ref[0]

    sblk_c = s_ref[0, 0, pl.ds(ib * _NB, _NB)][:, None]

    gi = (jnp.dot(base, Wg1_ref[0:D], preferred_element_type=jnp.float32)
          + jnp.dot(refined, Wg1_ref[D:2 * D],
                    preferred_element_type=jnp.float32)
          + sblk_c * Wg1_ref[2 * D:2 * D + 1]
          + bg1_ref[0])
    g = gi * jax.nn.sigmoid(gi)
    gate = jax.nn.sigmoid(
        jnp.sum(g * wg2t_ref[0][None, :], axis=1, keepdims=True)
        + bg2_ref[0, 0])

    gidx = ib * _NB + jax.lax.broadcasted_iota(jnp.int32, (_NB, 1), 0)
    nchunk = CC // 128

    def body(c, acc):
        sc = s_ref[0, 0, pl.ds(c * 128, 128)][None, :]
        cidx = c * 128 + jax.lax.broadcasted_iota(jnp.int32, (1, 128), 1)
        beats = (sc > sblk_c) | ((sc == sblk_c) & (cidx < gidx))
        return acc + jnp.sum(beats.astype(jnp.float32), axis=1, keepdims=True)

    rank = jax.lax.fori_loop(0, nchunk, body, jnp.zeros((_NB, 1), jnp.float32))
    mask = (rank < np.float32(k_sel)).astype(jnp.float32)

    out_row = base + (mask * gate) * (refined - base)
    out_ref[0] = out_row.T


def kernel(base_tokens, highres_tokens, selection_scores, W1, b1, W2, b2,
           Wg1, bg1, Wg2, bg2):
    B, CC, D = base_tokens.shape
    k_sel = max(1, int(round(CC * 0.15)))

    baseT = jnp.swapaxes(base_tokens, 1, 2)
    hrT = jnp.swapaxes(highres_tokens, 1, 2)
    s3 = selection_scores.reshape(B, 1, CC)
    b1r = b1.reshape(1, -1)
    b2r = b2.reshape(1, -1)
    bg1r = bg1.reshape(1, -1)
    wg2t = Wg2.reshape(1, -1)
    bg2r = bg2.reshape(1, 1)

    nblk = CC // _NB
    grid = (B, nblk)

    full = lambda shape: pl.BlockSpec(shape, lambda b, i: (0,) * len(shape))

    outT = pl.pallas_call(
        functools.partial(_fused_body, k_sel, CC),
        grid=grid,
        in_specs=[
            pl.BlockSpec((1, D, _NB), lambda b, i: (b, 0, i)),
            pl.BlockSpec((1, D, 4 * _NB), lambda b, i: (b, 0, i)),
            pl.BlockSpec((1, 1, CC), lambda b, i: (b, 0, 0)),
            full((4 * D, W1.shape[1])),
            full((1, b1.shape[0])),
            full(W2.shape),
            full((1, b2.shape[0])),
            full(Wg1.shape),
            full((1, bg1.shape[0])),
            full((1, Wg2.shape[0])),
            full((1, 1)),
        ],
        out_specs=pl.BlockSpec((1, D, _NB), lambda b, i: (b, 0, i)),
        out_shape=jax.ShapeDtypeStruct((B, D, CC), jnp.float32),
        compiler_params=pltpu.CompilerParams(
            dimension_semantics=("parallel", "parallel")),
    )(baseT, hrT, s3, W1, b1r, W2, b2r, Wg1, bg1r, wg2t, bg2r)
    return jnp.swapaxes(outT, 1, 2)


# speedup vs baseline: 1.9756x; 1.4408x over previous
import functools
import numpy as np
import jax
import jax.numpy as jnp
from jax.experimental import pallas as pl
from jax.experimental.pallas import tpu as pltpu

_NB = 512


def _fused_body(k_sel, CC, base_ref, hr_ref, s_ref, W1_ref, b1_ref, W2_ref,
                b2_ref, Wg1_ref, bg1_ref, wg2t_ref, bg2_ref, out_ref):
    ib = pl.program_id(1)
    D = base_ref.shape[1]

    bT = base_ref[0]
    base = bT.T
    H = hr_ref[0].T
    ni = _NB // 32
    H5 = H.reshape(ni, 2, 32, 2, D)
    pre = b1_ref[0]
    for p in range(2):
        for q in range(2):
            hpq = H5[:, p, :, q, :].reshape(_NB, D)
            Wpq = W1_ref[(2 * p + q) * D:(2 * p + q + 1) * D]
            pre = pre + jnp.dot(hpq, Wpq, preferred_element_type=jnp.float32)
    h = pre * 0.5 * (1.0 + jax.lax.erf(pre * np.float32(1.0 / np.sqrt(2.0))))
    refined = jnp.dot(h, W2_ref[...],
                      preferred_element_type=jnp.float32) + b2_ref[0]

    sblk_c = s_ref[0, 0, pl.ds(ib * _NB, _NB)][:, None]

    gi = (jnp.dot(base, Wg1_ref[0:D], preferred_element_type=jnp.float32)
          + jnp.dot(refined, Wg1_ref[D:2 * D],
                    preferred_element_type=jnp.float32)
          + sblk_c * Wg1_ref[2 * D:2 * D + 1]
          + bg1_ref[0])
    g = gi * jax.nn.sigmoid(gi)
    gate = jax.nn.sigmoid(
        jnp.sum(g * wg2t_ref[0][None, :], axis=1, keepdims=True)
        + bg2_ref[0, 0])

    gidx = ib * _NB + jax.lax.broadcasted_iota(jnp.int32, (_NB, 1), 0)
    nchunk = CC // 128

    def body(c, acc):
        sc = s_ref[0, 0, pl.ds(c * 128, 128)][None, :]
        cidx = c * 128 + jax.lax.broadcasted_iota(jnp.int32, (1, 128), 1)
        beats = (sc > sblk_c) | ((sc == sblk_c) & (cidx < gidx))
        return acc + jnp.sum(beats.astype(jnp.float32), axis=1, keepdims=True)

    rank = jax.lax.fori_loop(0, nchunk, body, jnp.zeros((_NB, 1), jnp.float32))
    mask = (rank < np.float32(k_sel)).astype(jnp.float32)

    out_row = base + (mask * gate) * (refined - base)
    out_ref[0] = out_row.T


def kernel(base_tokens, highres_tokens, selection_scores, W1, b1, W2, b2,
           Wg1, bg1, Wg2, bg2):
    B, CC, D = base_tokens.shape
    k_sel = max(1, int(round(CC * 0.15)))

    baseT = jnp.swapaxes(base_tokens, 1, 2)
    hrT = jnp.swapaxes(highres_tokens, 1, 2)
    s3 = selection_scores.reshape(B, 1, CC)
    b1r = b1.reshape(1, -1)
    b2r = b2.reshape(1, -1)
    bg1r = bg1.reshape(1, -1)
    wg2t = Wg2.reshape(1, -1)
    bg2r = bg2.reshape(1, 1)

    nblk = CC // _NB
    grid = (B, nblk)

    full = lambda shape: pl.BlockSpec(shape, lambda b, i: (0,) * len(shape))

    outT = pl.pallas_call(
        functools.partial(_fused_body, k_sel, CC),
        grid=grid,
        in_specs=[
            pl.BlockSpec((1, D, _NB), lambda b, i: (b, 0, i)),
            pl.BlockSpec((1, D, 4 * _NB), lambda b, i: (b, 0, i)),
            pl.BlockSpec((1, 1, CC), lambda b, i: (b, 0, 0)),
            full((4 * D, W1.shape[1])),
            full((1, b1.shape[0])),
            full(W2.shape),
            full((1, b2.shape[0])),
            full(Wg1.shape),
            full((1, bg1.shape[0])),
            full((1, Wg2.shape[0])),
            full((1, 1)),
        ],
        out_specs=pl.BlockSpec((1, D, _NB), lambda b, i: (b, 0, i)),
        out_shape=jax.ShapeDtypeStruct((B, D, CC), jnp.float32),
        compiler_params=pltpu.CompilerParams(
            dimension_semantics=("parallel", "parallel")),
    )(baseT, hrT, s3, W1, b1r, W2, b2r, Wg1, bg1r, wg2t, bg2r)
    return jnp.swapaxes(outT, 1, 2)


# speedup vs baseline: 2.3926x; 1.2111x over previous
import functools
import numpy as np
import jax
import jax.numpy as jnp
from jax.experimental import pallas as pl
from jax.experimental.pallas import tpu as pltpu

_NB = 1024


def _fused_body(k_sel, CC, base_ref, hr_ref, s_ref, W1_ref, b1_ref, W2_ref,
                b2_ref, Wg1_ref, bg1_ref, wg2t_ref, bg2_ref, out_ref):
    ib = pl.program_id(1)
    D = base_ref.shape[1]

    bT = base_ref[0]
    base = bT.T
    H = hr_ref[0].T
    ni = _NB // 32
    H5 = H.reshape(ni, 2, 32, 2, D)
    pre = b1_ref[0]
    for p in range(2):
        for q in range(2):
            hpq = H5[:, p, :, q, :].reshape(_NB, D)
            Wpq = W1_ref[(2 * p + q) * D:(2 * p + q + 1) * D]
            pre = pre + jnp.dot(hpq, Wpq, preferred_element_type=jnp.float32)
    h = pre * 0.5 * (1.0 + jax.lax.erf(pre * np.float32(1.0 / np.sqrt(2.0))))
    refined = jnp.dot(h, W2_ref[...],
                      preferred_element_type=jnp.float32) + b2_ref[0]

    sblk_c = s_ref[0, 0, pl.ds(ib * _NB, _NB)][:, None]

    gi = (jnp.dot(base, Wg1_ref[0:D], preferred_element_type=jnp.float32)
          + jnp.dot(refined, Wg1_ref[D:2 * D],
                    preferred_element_type=jnp.float32)
          + sblk_c * Wg1_ref[2 * D:2 * D + 1]
          + bg1_ref[0])
    g = gi * jax.nn.sigmoid(gi)
    gate = jax.nn.sigmoid(
        jnp.sum(g * wg2t_ref[0][None, :], axis=1, keepdims=True)
        + bg2_ref[0, 0])

    gidx = ib * _NB + jax.lax.broadcasted_iota(jnp.int32, (_NB, 1), 0)
    nchunk = CC // 128

    def body(c, acc):
        sc = s_ref[0, 0, pl.ds(c * 128, 128)][None, :]
        cidx = c * 128 + jax.lax.broadcasted_iota(jnp.int32, (1, 128), 1)
        beats = (sc > sblk_c) | ((sc == sblk_c) & (cidx < gidx))
        return acc + jnp.sum(beats.astype(jnp.float32), axis=1, keepdims=True)

    rank = jax.lax.fori_loop(0, nchunk, body, jnp.zeros((_NB, 1), jnp.float32))
    mask = (rank < np.float32(k_sel)).astype(jnp.float32)

    out_row = base + (mask * gate) * (refined - base)
    out_ref[0] = out_row.T


def kernel(base_tokens, highres_tokens, selection_scores, W1, b1, W2, b2,
           Wg1, bg1, Wg2, bg2):
    B, CC, D = base_tokens.shape
    k_sel = max(1, int(round(CC * 0.15)))

    baseT = jnp.swapaxes(base_tokens, 1, 2)
    hrT = jnp.swapaxes(highres_tokens, 1, 2)
    s3 = selection_scores.reshape(B, 1, CC)
    b1r = b1.reshape(1, -1)
    b2r = b2.reshape(1, -1)
    bg1r = bg1.reshape(1, -1)
    wg2t = Wg2.reshape(1, -1)
    bg2r = bg2.reshape(1, 1)

    nblk = CC // _NB
    grid = (B, nblk)

    full = lambda shape: pl.BlockSpec(shape, lambda b, i: (0,) * len(shape))

    outT = pl.pallas_call(
        functools.partial(_fused_body, k_sel, CC),
        grid=grid,
        in_specs=[
            pl.BlockSpec((1, D, _NB), lambda b, i: (b, 0, i)),
            pl.BlockSpec((1, D, 4 * _NB), lambda b, i: (b, 0, i)),
            pl.BlockSpec((1, 1, CC), lambda b, i: (b, 0, 0)),
            full((4 * D, W1.shape[1])),
            full((1, b1.shape[0])),
            full(W2.shape),
            full((1, b2.shape[0])),
            full(Wg1.shape),
            full((1, bg1.shape[0])),
            full((1, Wg2.shape[0])),
            full((1, 1)),
        ],
        out_specs=pl.BlockSpec((1, D, _NB), lambda b, i: (b, 0, i)),
        out_shape=jax.ShapeDtypeStruct((B, D, CC), jnp.float32),
        compiler_params=pltpu.CompilerParams(
            dimension_semantics=("parallel", "parallel")),
    )(baseT, hrT, s3, W1, b1r, W2, b2r, Wg1, bg1r, wg2t, bg2r)
    return jnp.swapaxes(outT, 1, 2)
